# baseline (speedup 1.0000x reference)
import functools
import math

import jax
import jax.numpy as jnp
from jax.experimental import pallas as pl
from jax.experimental.pallas import tpu as pltpu

_B, _H, _QL, _KL, _D = 8, 32, 4, 2048, 128
_T = 512
_HH = _H // 2


def _flash_body(q_ref, k0_ref, k1_ref, v0_ref, v1_ref, o_ref,
                acc_ref, m_ref, l_ref):
    c = pl.program_id(1)
    nc = pl.num_programs(1)

    @pl.when(c == 0)
    def _init():
        m_ref[...] = jnp.full_like(m_ref, -jnp.inf)
        l_ref[...] = jnp.zeros_like(l_ref)
        acc_ref[...] = jnp.zeros_like(acc_ref)

    k0 = k0_ref[0]
    k1 = k1_ref[0]
    v0 = v0_ref[0]
    v1 = v1_ref[0]
    q = q_ref[0]

    kmin = jnp.minimum(jnp.min(k0, axis=0), jnp.min(k1, axis=0))
    kmax = jnp.maximum(jnp.max(k0, axis=0), jnp.max(k1, axis=0))
    ks = (kmax - kmin) * (1.0 / 3.0)
    ks = jnp.where(ks == 0, 1.0, ks)
    kinv = 1.0 / ks
    kb = -kmin * kinv
    kd0 = jnp.round(k0 * kinv + kb) * ks + kmin
    kd1 = jnp.round(k1 * kinv + kb) * ks + kmin

    vm0 = jnp.minimum(jnp.min(v0, axis=0), jnp.min(v1, axis=0))
    vx0 = jnp.maximum(jnp.max(v0, axis=0), jnp.max(v1, axis=0))
    vmin = jnp.min(vm0, axis=1)
    vmax = jnp.max(vx0, axis=1)
    vs = (vmax - vmin) * (1.0 / 15.0)
    vs = jnp.where(vs == 0, 1.0, vs)
    vinv = 1.0 / vs
    vb = -vmin * vinv
    vd0 = (jnp.round(v0 * vinv[None, :, None] + vb[None, :, None])
           * vs[None, :, None] + vmin[None, :, None])
    vd1 = (jnp.round(v1 * vinv[None, :, None] + vb[None, :, None])
           * vs[None, :, None] + vmin[None, :, None])

    scale = 1.0 / math.sqrt(float(_D))
    dn = (((2,), (2,)), ((0,), (0,)))
    s0 = jax.lax.dot_general(q[:_HH], kd0, dn,
                             preferred_element_type=jnp.float32)
    s1 = jax.lax.dot_general(q[_HH:], kd1, dn,
                             preferred_element_type=jnp.float32)
    s = jnp.concatenate([s0, s1], axis=0) * scale

    m_prev = m_ref[...]
    m_new = jnp.maximum(m_prev, jnp.max(s, axis=2))
    alpha = jnp.exp(m_prev - m_new)
    p = jnp.exp(s - m_new[..., None])
    l_ref[...] = l_ref[...] * alpha + jnp.sum(p, axis=2)
    dn_pv = (((2,), (1,)), ((0,), (0,)))
    pv0 = jax.lax.dot_general(p[:_HH], vd0, dn_pv,
                              preferred_element_type=jnp.float32)
    pv1 = jax.lax.dot_general(p[_HH:], vd1, dn_pv,
                              preferred_element_type=jnp.float32)
    pv = jnp.concatenate([pv0, pv1], axis=0)
    acc_ref[...] = acc_ref[...] * alpha[..., None] + pv
    m_ref[...] = m_new

    @pl.when(c == nc - 1)
    def _flush():
        o_ref[0] = acc_ref[...] / l_ref[...][..., None]


@jax.jit
def kernel(query, key, value):
    nc = _KL // _T
    grid = (_B, nc)
    kv_spec_lo = pl.BlockSpec((1, _HH, _T, _D), lambda b, c: (b, 0, c, 0))
    kv_spec_hi = pl.BlockSpec((1, _HH, _T, _D), lambda b, c: (b, 1, c, 0))
    out = pl.pallas_call(
        _flash_body,
        grid=grid,
        in_specs=[
            pl.BlockSpec((1, _H, _QL, _D), lambda b, c: (b, 0, 0, 0)),
            kv_spec_lo,
            kv_spec_hi,
            pl.BlockSpec((1, _HH, _T, _D), lambda b, c: (b, 0, c, 0)),
            pl.BlockSpec((1, _HH, _T, _D), lambda b, c: (b, 1, c, 0)),
        ],
        out_specs=pl.BlockSpec((1, _H, _QL, _D), lambda b, c: (b, 0, 0, 0)),
        out_shape=jax.ShapeDtypeStruct((_B, _H, _QL, _D), jnp.float32),
        scratch_shapes=[
            pltpu.VMEM((_H, _QL, _D), jnp.float32),
            pltpu.VMEM((_H, _QL), jnp.float32),
            pltpu.VMEM((_H, _QL), jnp.float32),
        ],
        compiler_params=pltpu.CompilerParams(
            dimension_semantics=("parallel", "arbitrary"),
        ),
    )(query, key, key, value, value)
    return out


# speedup vs baseline: 1.0448x; 1.0448x over previous
import functools
import math

import jax
import jax.numpy as jnp
from jax.experimental import pallas as pl
from jax.experimental.pallas import tpu as pltpu

_B, _H, _QL, _KL, _D = 8, 32, 4, 2048, 128
_T = 512


def _minmax_over_heads(x):
    mn = x[0]
    mx = x[0]
    for h in range(1, x.shape[0]):
        t = x[h]
        mn = jnp.minimum(mn, t)
        mx = jnp.maximum(mx, t)
    return mn, mx


def _flash_body(q_ref, k_ref, v_ref, o_ref, acc_ref, m_ref, l_ref):
    c = pl.program_id(1)
    nc = pl.num_programs(1)

    @pl.when(c == 0)
    def _init():
        m_ref[...] = jnp.full_like(m_ref, -jnp.inf)
        l_ref[...] = jnp.zeros_like(l_ref)
        acc_ref[...] = jnp.zeros_like(acc_ref)

    k = k_ref[0]
    v = v_ref[0]
    q = q_ref[0]

    kmin, kmax = _minmax_over_heads(k)
    ks = (kmax - kmin) * (1.0 / 3.0)
    ks = jnp.where(ks == 0, 1.0, ks)
    kinv = 1.0 / ks
    kb = -kmin * kinv
    kd = jnp.round(k * kinv + kb) * ks + kmin

    vm0, vx0 = _minmax_over_heads(v)
    vmin = jnp.min(vm0, axis=1)
    vmax = jnp.max(vx0, axis=1)
    vs = (vmax - vmin) * (1.0 / 15.0)
    vs = jnp.where(vs == 0, 1.0, vs)
    vinv = 1.0 / vs
    vb = -vmin * vinv
    vc = jnp.round(v * vinv[None, :, None] + vb[None, :, None]) - 7.5

    scale = 1.0 / math.sqrt(float(_D))
    s = jax.lax.dot_general(
        q, kd, (((2,), (2,)), ((0,), (0,))),
        preferred_element_type=jnp.float32,
    ) * scale

    m_prev = m_ref[...]
    m_new = jnp.maximum(m_prev, jnp.max(s, axis=2))
    alpha = jnp.exp(m_prev - m_new)
    p = jnp.exp(s - m_new[..., None])
    l_ref[...] = l_ref[...] * alpha + jnp.sum(p, axis=2)
    pv = jax.lax.dot_general(
        p * vs[None, None, :], vc, (((2,), (1,)), ((0,), (0,))),
        preferred_element_type=jnp.float32,
    )
    corrvec = vmin + 7.5 * vs
    corr = jnp.sum(p * corrvec[None, None, :], axis=2)
    acc_ref[...] = acc_ref[...] * alpha[..., None] + pv + corr[..., None]
    m_ref[...] = m_new

    @pl.when(c == nc - 1)
    def _flush():
        o_ref[0] = acc_ref[...] / l_ref[...][..., None]


@jax.jit
def kernel(query, key, value):
    nc = _KL // _T
    grid = (_B, nc)
    out = pl.pallas_call(
        _flash_body,
        grid=grid,
        in_specs=[
            pl.BlockSpec((1, _H, _QL, _D), lambda b, c: (b, 0, 0, 0)),
            pl.BlockSpec((1, _H, _T, _D), lambda b, c: (b, 0, c, 0)),
            pl.BlockSpec((1, _H, _T, _D), lambda b, c: (b, 0, c, 0)),
        ],
        out_specs=pl.BlockSpec((1, _H, _QL, _D), lambda b, c: (b, 0, 0, 0)),
        out_shape=jax.ShapeDtypeStruct((_B, _H, _QL, _D), jnp.float32),
        scratch_shapes=[
            pltpu.VMEM((_H, _QL, _D), jnp.float32),
            pltpu.VMEM((_H, _QL), jnp.float32),
            pltpu.VMEM((_H, _QL), jnp.float32),
        ],
        compiler_params=pltpu.CompilerParams(
            dimension_semantics=("parallel", "arbitrary"),
        ),
    )(query, key, value)
    return out
